# baseline (speedup 1.0000x reference)
import functools

import jax
import jax.numpy as jnp
from jax.experimental import pallas as pl

NEG_INF = -1000000000.0
_BR = 64


def _softmax3_block(x_ref, o_ref, *, n, block_rows):
    i = pl.program_id(0)
    a0 = x_ref[0]
    a1 = x_ref[1]
    a2 = x_ref[2]
    row = jax.lax.broadcasted_iota(jnp.int32, a0.shape, 0) + i * block_rows
    col = jax.lax.broadcasted_iota(jnp.int32, a0.shape, 1)
    a0 = jnp.where(row == n - 1, NEG_INF, a0)
    a1 = jnp.where(col == n - 1, NEG_INF, a1)
    m = jnp.maximum(jnp.maximum(a0, a1), a2)
    e0 = jnp.exp(a0 - m)
    e1 = jnp.exp(a1 - m)
    e2 = jnp.exp(a2 - m)
    inv = 1.0 / (e0 + e1 + e2)
    o_ref[0] = e0 * inv
    o_ref[1] = e1 * inv
    o_ref[2] = e2 * inv


def kernel(log_edge_flows):
    n = log_edge_flows.shape[0]
    x = jnp.transpose(log_edge_flows, (2, 0, 1))
    grid = (pl.cdiv(n, _BR),)
    out = pl.pallas_call(
        functools.partial(_softmax3_block, n=n, block_rows=_BR),
        grid=grid,
        in_specs=[pl.BlockSpec((3, _BR, n), lambda i: (0, i, 0))],
        out_specs=pl.BlockSpec((3, _BR, n), lambda i: (0, i, 0)),
        out_shape=jax.ShapeDtypeStruct((3, n, n), jnp.float32),
    )(x)
    return jnp.transpose(out, (1, 2, 0))


# speedup vs baseline: 1.0660x; 1.0660x over previous
import functools

import jax
import jax.numpy as jnp
from jax.experimental import pallas as pl

NEG_INF = -1000000000.0
_BR = 224


def _softmax3_block(x_ref, o_ref, *, n, block_rows):
    i = pl.program_id(0)
    a0 = x_ref[0]
    a1 = x_ref[1]
    a2 = x_ref[2]
    row = jax.lax.broadcasted_iota(jnp.int32, a0.shape, 0) + i * block_rows
    col = jax.lax.broadcasted_iota(jnp.int32, a0.shape, 1)
    a0 = jnp.where(row == n - 1, NEG_INF, a0)
    a1 = jnp.where(col == n - 1, NEG_INF, a1)
    m = jnp.maximum(jnp.maximum(a0, a1), a2)
    e0 = jnp.exp(a0 - m)
    e1 = jnp.exp(a1 - m)
    e2 = jnp.exp(a2 - m)
    inv = 1.0 / (e0 + e1 + e2)
    o_ref[0] = e0 * inv
    o_ref[1] = e1 * inv
    o_ref[2] = e2 * inv


def kernel(log_edge_flows):
    n = log_edge_flows.shape[0]
    x = jnp.transpose(log_edge_flows, (2, 0, 1))
    grid = (pl.cdiv(n, _BR),)
    out = pl.pallas_call(
        functools.partial(_softmax3_block, n=n, block_rows=_BR),
        grid=grid,
        in_specs=[pl.BlockSpec((3, _BR, n), lambda i: (0, i, 0))],
        out_specs=pl.BlockSpec((3, _BR, n), lambda i: (0, i, 0)),
        out_shape=jax.ShapeDtypeStruct((3, n, n), jnp.float32),
    )(x)
    return jnp.transpose(out, (1, 2, 0))
